# Initial kernel scaffold; baseline (speedup 1.0000x reference)
import jax
import jax.numpy as jnp
from jax.experimental import pallas as pl


def kernel(x, code_book):
    raise NotImplementedError("write your pallas kernel here")


# speedup vs baseline: 1.3371x; 1.3371x over previous
import functools

import jax
import jax.numpy as jnp
from jax import lax
from jax.experimental import pallas as pl
from jax.experimental.pallas import tpu as pltpu
from jax.experimental.pallas import tpu_sc as plsc

NUM_EMB = 8192
DIM = 32
BETA = 0.25
ROWS = 16384
BLK = 512

_NC = 2
_NS = 16
_NW = _NC * _NS
_BPW = ROWS // _NW
_LANES = 16


def _argmin_body(xn_ref, x_ref, cb_ref, cbn_ref, ids_ref, dmin_ref):
    m = lax.dot_general(x_ref[...], cb_ref[...],
                        (((1,), (1,)), ((), ())),
                        preferred_element_type=jnp.float32)
    d = cbn_ref[...][None, :] + xn_ref[...][:, None] - 2.0 * m
    dmin = jnp.min(d, axis=-1, keepdims=True)
    jidx = lax.broadcasted_iota(jnp.int32, d.shape, 1)
    ids_ref[...] = jnp.min(jnp.where(d == dmin, jidx, NUM_EMB), axis=-1)
    dmin_ref[...] = dmin[:, 0]


def _argmin_call(xn, x_, code_book, cbn):
    grid = (ROWS // BLK,)
    return pl.pallas_call(
        _argmin_body,
        grid=grid,
        in_specs=[
            pl.BlockSpec((BLK,), lambda i: (i,)),
            pl.BlockSpec((BLK, DIM), lambda i: (i, 0)),
            pl.BlockSpec((NUM_EMB, DIM), lambda i: (0, 0)),
            pl.BlockSpec((NUM_EMB,), lambda i: (0,)),
        ],
        out_specs=[
            pl.BlockSpec((BLK,), lambda i: (i,)),
            pl.BlockSpec((BLK,), lambda i: (i,)),
        ],
        out_shape=[
            jax.ShapeDtypeStruct((ROWS,), jnp.int32),
            jax.ShapeDtypeStruct((ROWS,), jnp.float32),
        ],
    )(xn, x_, code_book, cbn)


def _sc_gather(code_book, ids_flat, dmin):
    mesh = plsc.VectorSubcoreMesh(core_axis_name="c", subcore_axis_name="s")

    @functools.partial(
        pl.kernel,
        mesh=mesh,
        out_type=[
            jax.ShapeDtypeStruct((ROWS, DIM), jnp.float32),
            jax.ShapeDtypeStruct((_NW, _LANES), jnp.float32),
        ],
        scratch_types=[
            pltpu.VMEM((_BPW,), jnp.int32),
            pltpu.VMEM((_BPW, DIM), jnp.float32),
            pltpu.VMEM((_BPW,), jnp.float32),
            pltpu.VMEM((_LANES,), jnp.float32),
            pltpu.SemaphoreType.DMA,
        ],
        compiler_params=pltpu.CompilerParams(use_tc_tiling_on_sc=False),
    )
    def gather(table_hbm, idx_hbm, dmin_hbm, out_hbm, lsum_hbm,
               idx_v, rows_v, dmin_v, acc_v, sem):
        wid = lax.axis_index("s") * _NC + lax.axis_index("c")
        base = wid * _BPW
        pltpu.sync_copy(idx_hbm.at[pl.ds(base, _BPW)], idx_v)
        cp = pltpu.async_copy(table_hbm.at[idx_v], rows_v, sem)
        pltpu.sync_copy(dmin_hbm.at[pl.ds(base, _BPW)], dmin_v)
        acc_v[...] = jnp.zeros((_LANES,), jnp.float32)

        def body(i, carry):
            acc_v[...] = acc_v[...] + dmin_v[pl.ds(i * _LANES, _LANES)]
            return carry

        lax.fori_loop(0, _BPW // _LANES, body, 0)
        pltpu.sync_copy(acc_v, lsum_hbm.at[wid])
        cp.wait()
        pltpu.sync_copy(rows_v, out_hbm.at[pl.ds(base, _BPW)])

    return gather(code_book, ids_flat, dmin)


def kernel(x, code_book):
    b, c, h, w = x.shape
    x_ = jnp.transpose(x, (0, 2, 3, 1)).reshape(-1, c)
    xn = jnp.sum(x_ ** 2, axis=-1)
    cbn = jnp.sum(code_book ** 2, axis=-1)
    ids_flat, dmin = _argmin_call(xn, x_, code_book, cbn)
    emb_flat, lsum = _sc_gather(code_book, ids_flat, dmin)
    ids = ids_flat.reshape(b, h, w)
    emb_st = jnp.transpose(emb_flat.reshape(b, h, w, c), (0, 3, 1, 2))
    emb_loss = (1.0 + BETA) * jnp.sum(lsum) / (b * c * h * w)
    return (ids, emb_st, emb_loss)
